# baseline (speedup 1.0000x reference)
import jax
import jax.numpy as jnp
from jax.experimental import pallas as pl
from jax.experimental.pallas import tpu as pltpu

_LAMBDA = 1.0
_NEG_INF = float("-inf")
_B = 4096
_GC = 1024


def _moe_dense_kernel(x_ref, wg_ref, bg_ref, w_ref, b_ref, out_ref, cv_ref,
                      p_scratch):
    e = pl.program_id(0)

    @pl.when(e == 0)
    def _gate():
        for c in range(_B // _GC):
            xv = x_ref[c * _GC:(c + 1) * _GC, :]
            logits = jax.lax.dot_general(
                xv, wg_ref[...], (((1,), (1,)), ((), ())),
                preferred_element_type=jnp.float32) + bg_ref[...]
            idx8 = jax.lax.broadcasted_iota(jnp.int32, (_GC, 8), 1)
            m1 = jnp.max(logits, axis=1, keepdims=True)
            i1 = jnp.min(jnp.where(logits == m1, idx8, 8), axis=1,
                         keepdims=True)
            sel1 = idx8 == i1
            masked = jnp.where(sel1, _NEG_INF, logits)
            m2 = jnp.max(masked, axis=1, keepdims=True)
            i2 = jnp.min(jnp.where(masked == m2, idx8, 8), axis=1,
                         keepdims=True)
            sel2 = idx8 == i2
            e2 = jnp.exp(m2 - m1)
            z = 1.0 + e2
            p1 = 1.0 / z
            p2 = e2 / z
            p_scratch[c * _GC:(c + 1) * _GC, :] = jnp.where(
                sel1, p1, jnp.where(sel2, p2, 0.0))
        load = jnp.sum(p_scratch[...], axis=0, keepdims=True)
        mean = jnp.sum(load) / 8.0
        var = jnp.sum((load - mean) ** 2) / 7.0
        cv = jnp.sqrt(var) / mean
        cv_ref[...] = jnp.full((8, 128), cv, jnp.float32)

    wb = w_ref[0].astype(jnp.bfloat16)
    for c in range(_B // _GC):
        rows = pl.ds(c * _GC, _GC)
        pe = jnp.sum(
            jnp.where(
                jax.lax.broadcasted_iota(jnp.int32, (_GC, 8), 1) == e,
                p_scratch[rows, :], 0.0),
            axis=1, keepdims=True)
        xb = x_ref[rows, :].astype(jnp.bfloat16)
        y = jax.lax.dot_general(
            xb, wb, (((1,), (1,)), ((), ())),
            preferred_element_type=jnp.float32) + b_ref[0]
        contrib = pe * y

        @pl.when(e == 0)
        def _init():
            out_ref[rows, :] = contrib

        @pl.when(e != 0)
        def _acc():
            out_ref[rows, :] += contrib


def kernel(x, W_experts, b_experts, W_gate, b_gate):
    out, cvb = pl.pallas_call(
        _moe_dense_kernel,
        grid=(8,),
        in_specs=[
            pl.BlockSpec((_B, 1024), lambda e: (0, 0)),
            pl.BlockSpec((8, 1024), lambda e: (0, 0)),
            pl.BlockSpec((1, 8), lambda e: (0, 0)),
            pl.BlockSpec((1, 1024, 1024), lambda e: (e, 0, 0)),
            pl.BlockSpec((1, 1, 1024), lambda e: (e, 0, 0)),
        ],
        out_specs=[
            pl.BlockSpec((_B, 1024), lambda e: (0, 0)),
            pl.BlockSpec((8, 128), lambda e: (0, 0)),
        ],
        out_shape=[
            jax.ShapeDtypeStruct((_B, 1024), jnp.float32),
            jax.ShapeDtypeStruct((8, 128), jnp.float32),
        ],
        scratch_shapes=[
            pltpu.VMEM((_B, 8), jnp.float32),
        ],
    )(x, W_gate, b_gate.reshape(1, 8), W_experts,
      b_experts.reshape(8, 1, 1024))
    return (out, _LAMBDA * cvb[0, 0])
